# baseline (speedup 1.0000x reference)
import functools

import jax
import jax.numpy as jnp
from jax import lax
from jax.experimental import pallas as pl
from jax.experimental.pallas import tpu as pltpu
from jax.experimental.pallas import tpu_sc as plsc

D_EMB = 128
NUM_WORKERS = 32
CHUNK = 80
U = 5
ROT = 4


def _bond_encode_sc(ea_packed, table, E, n1, n2):
    per_w = E // NUM_WORKERS
    n_rows = 60
    n_bodies = per_w // (CHUNK * U)
    mesh = plsc.VectorSubcoreMesh(core_axis_name="c", subcore_axis_name="s")

    scratch = (
        [pltpu.VMEM((U, 1, 3, CHUNK), jnp.int32)]
        + [pltpu.VMEM((U, CHUNK), jnp.int32)]
        + [pltpu.VMEM((U, CHUNK, D_EMB), jnp.float32)]
        + [pltpu.SemaphoreType.DMA for _ in range(3 * U)]
    )

    @functools.partial(
        pl.kernel,
        mesh=mesh,
        out_type=jax.ShapeDtypeStruct((E, D_EMB), jnp.float32),
        scratch_types=scratch,
    )
    def k(ea_h, s_h, out_h, a_v, idx_v, rows_v, *sems):
        sem_in = sems[0:U]
        sem_g = sems[U:2 * U]
        sem_cp = sems[2 * U:3 * U]
        wid = lax.axis_index("s") * 2 + lax.axis_index("c")
        base_w = wid * per_w
        row_w = wid * (per_w // CHUNK)

        def body(bi, carry):
            base0 = base_w + bi * (CHUNK * U)
            row0 = row_w + bi * U
            rep0 = lax.rem(bi, ROT) * (NUM_WORKERS * U) + wid * U
            ins = []
            for u in range(U):
                ins.append(pltpu.async_copy(
                    ea_h.at[pl.ds(row0 + u, 1)], a_v.at[u], sem_in[u]))
            gs = []
            for u in range(U):
                ins[u].wait()
                off = (rep0 + u) * n_rows
                for i in range(CHUNK // 16):
                    s = pl.ds(i * 16, 16)
                    idx_v[u, s] = (a_v[u, 0, 0, s] * (n1 * n2)
                                   + a_v[u, 0, 1, s] * n2
                                   + a_v[u, 0, 2, s] + off)
                gs.append(pltpu.async_copy(s_h.at[idx_v.at[u]], rows_v.at[u], sem_g[u]))
            cps = []
            for u in range(U):
                gs[u].wait()
                base = base0 + u * CHUNK
                cps.append(pltpu.async_copy(
                    rows_v.at[u], out_h.at[pl.ds(base, CHUNK)], sem_cp[u]))
            for u in range(U):
                cps[u].wait()
            return carry

        lax.fori_loop(0, n_bodies, body, 0)

    return k(ea_packed, table)


def kernel(edge_attr, W0, W1, W2):
    E = edge_attr.shape[0]
    n0, n1, n2 = W0.shape[0], W1.shape[0], W2.shape[0]
    table = (W0[:, None, None, :] + W1[None, :, None, :]
             + W2[None, None, :, :]).reshape(n0 * n1 * n2, D_EMB)
    table = jnp.tile(table, (NUM_WORKERS * U * ROT, 1))
    ea_packed = edge_attr.reshape(E // CHUNK, CHUNK, 3).transpose(0, 2, 1)
    return _bond_encode_sc(ea_packed, table, E, n1, n2)


# speedup vs baseline: 2.6248x; 2.6248x over previous
import functools

import jax
import jax.numpy as jnp
from jax import lax
from jax.experimental import pallas as pl
from jax.experimental.pallas import tpu as pltpu
from jax.experimental.pallas import tpu_sc as plsc

D_EMB = 128
NUM_WORKERS = 32
CHUNK = 80
U = 5


def _bond_encode_sc(codes0, codes1, codes2, table, E, n1, n2):
    per_w = E // NUM_WORKERS
    n_bodies = per_w // (CHUNK * U)
    mesh = plsc.VectorSubcoreMesh(core_axis_name="c", subcore_axis_name="s")

    scratch = (
        [pltpu.VMEM((U, CHUNK), jnp.int32) for _ in range(3)]
        + [pltpu.VMEM((U, CHUNK), jnp.int32)]
        + [pltpu.VMEM((U, CHUNK, D_EMB), jnp.float32)]
        + [pltpu.VMEM_SHARED((16 * 60, D_EMB), jnp.float32)]
        + [pltpu.SemaphoreType.DMA for _ in range(3 * U + 1)]
    )

    @functools.partial(
        pl.kernel,
        mesh=mesh,
        out_type=jax.ShapeDtypeStruct((E, D_EMB), jnp.float32),
        scratch_types=scratch,
    )
    def k(c0_h, c1_h, c2_h, s_h, out_h, a0_v, a1_v, a2_v, idx_v, rows_v, s_v, *sems):
        sem_in = sems[0:U]
        sem_g = sems[U:2 * U]
        sem_cp = sems[2 * U:3 * U]
        sid = lax.axis_index("s")
        wid = sid * 2 + lax.axis_index("c")
        base_w = wid * per_w
        pltpu.async_copy(s_h, s_v.at[pl.ds(sid * 60, 60)], sems[3 * U]).wait()
        plsc.subcore_barrier()

        def fire_in(base, u):
            for h, v in ((c0_h, a0_v), (c1_h, a1_v), (c2_h, a2_v)):
                pltpu.async_copy(h.at[pl.ds(base, CHUNK)], v.at[u], sem_in[u])

        def absorb_in(u):
            for h, v in ((c0_h, a0_v), (c1_h, a1_v), (c2_h, a2_v)):
                pltpu.make_async_copy(h.at[pl.ds(0, CHUNK)], v.at[u], sem_in[u]).wait()

        for u in range(U):
            fire_in(base_w + u * CHUNK, u)

        def body(bi, carry):
            base0 = base_w + bi * (CHUNK * U)
            gs = []
            for u in range(U):
                absorb_in(u)
                off = sid * 60
                for i in range(CHUNK // 16):
                    s = pl.ds(i * 16, 16)
                    idx_v[u, s] = (a0_v[u, s] * (n1 * n2) + a1_v[u, s] * n2
                                   + a2_v[u, s] + off)

                @pl.when(bi != n_bodies - 1)
                def _():
                    fire_in(base0 + (U + u) * CHUNK, u)

                @pl.when(bi != 0)
                def _():
                    pltpu.make_async_copy(
                        rows_v.at[u], out_h.at[pl.ds(0, CHUNK)], sem_cp[u]).wait()

                gs.append(pltpu.async_copy(
                    s_v.at[idx_v.at[u]], rows_v.at[u], sem_g[u]))
            for u in range(U):
                gs[u].wait()
                pltpu.async_copy(
                    rows_v.at[u], out_h.at[pl.ds(base0 + u * CHUNK, CHUNK)],
                    sem_cp[u])
            return carry

        lax.fori_loop(0, n_bodies, body, 0)
        for u in range(U):
            pltpu.make_async_copy(
                rows_v.at[u], out_h.at[pl.ds(0, CHUNK)], sem_cp[u]).wait()

    return k(codes0, codes1, codes2, table)


def kernel(edge_attr, W0, W1, W2):
    E = edge_attr.shape[0]
    n0, n1, n2 = W0.shape[0], W1.shape[0], W2.shape[0]
    table = (W0[:, None, None, :] + W1[None, :, None, :]
             + W2[None, None, :, :]).reshape(n0 * n1 * n2, D_EMB)
    return _bond_encode_sc(edge_attr[:, 0], edge_attr[:, 1], edge_attr[:, 2],
                           table, E, n1, n2)
